# baseline (speedup 1.0000x reference)
import functools

import jax
import jax.numpy as jnp
from jax import lax
from jax.experimental import pallas as pl
from jax.experimental.pallas import tpu as pltpu
from jax.experimental.pallas import tpu_sc as plsc

N = 10000
E = 160000
D = 128
NG = 8
SEG = N // NG

NC = 2
NS = 16
NW = NC * NS
K = 128
CPT = 40
EP = NW * CPT * K
NCH = EP // K
ACC_ROWS = N + 8
ROWS_PER_TILE = (N // NS) // 8 * 8
ROWS_TAIL = N - NS * ROWS_PER_TILE

_mesh = plsc.VectorSubcoreMesh(
    core_axis_name="c", subcore_axis_name="s", num_cores=NC, num_subcores=NS
)

_dn = lax.GatherDimensionNumbers(
    offset_dims=(), collapsed_slice_dims=(0,), start_index_map=(0,))


def _vgather(vals, idx):
    return lax.gather(vals, idx[:, None], _dn, slice_sizes=(1,),
                      mode=lax.GatherScatterMode.PROMISE_IN_BOUNDS)


@functools.partial(
    pl.kernel,
    out_type=jax.ShapeDtypeStruct((NC, N, D), jnp.float32),
    mesh=_mesh,
    scratch_types=[
        pltpu.VMEM((K,), jnp.int32),
        pltpu.VMEM((K,), jnp.int32),
        pltpu.VMEM((K,), jnp.int32),
        pltpu.VMEM((K, 2 * D), jnp.float32),
        pltpu.VMEM((K, D), jnp.float32),
        pltpu.VMEM((16,), jnp.float32),
        pltpu.VMEM((16,), jnp.float32),
        pltpu.VMEM_SHARED((ACC_ROWS, D), jnp.float32),
        pltpu.SemaphoreType.DMA,
    ],
)
def _sc_edge_pass(z_hbm, src_hbm, dst_hbm, rel_hbm, cab_hbm, zero_hbm,
                  part_hbm, src_v, dst_v, rel_v, rows_v, msg_v, ca_v, cb_v,
                  acc, sem):
    cid = lax.axis_index("c")
    sid = lax.axis_index("s")
    wid = sid * NC + cid
    pltpu.sync_copy(cab_hbm.at[0], ca_v)
    pltpu.sync_copy(cab_hbm.at[1], cb_v)
    row0 = sid * ROWS_PER_TILE
    pltpu.sync_copy(zero_hbm.at[pl.ds(row0, ROWS_PER_TILE)],
                    acc.at[pl.ds(row0, ROWS_PER_TILE)])

    @pl.when(sid == NS - 1)
    def _():
        pltpu.sync_copy(zero_hbm.at[pl.ds(NS * ROWS_PER_TILE, ROWS_TAIL)],
                        acc.at[pl.ds(NS * ROWS_PER_TILE, ROWS_TAIL)])

    plsc.subcore_barrier()

    def chunk(i, carry):
        eb = (i * NW + wid) * K
        pltpu.sync_copy(src_hbm.at[pl.ds(eb, K)], src_v)
        pltpu.sync_copy(dst_hbm.at[pl.ds(eb, K)], dst_v)
        pltpu.sync_copy(rel_hbm.at[pl.ds(eb, K)], rel_v)
        pltpu.async_copy(z_hbm.at[src_v], rows_v, sem).wait()

        def group(g, carry2):
            rel_vec = rel_v[pl.ds(g * 16, 16)]
            a_vec = _vgather(ca_v[...], rel_vec)
            b_vec = _vgather(cb_v[...], rel_vec)
            for k in range(16):
                a = a_vec[k]
                b = b_vec[k]
                e = g * 16 + k
                for j in range(D // 16):
                    v0 = rows_v[e, pl.ds(j * 16, 16)]
                    v1 = rows_v[e, pl.ds(D + j * 16, 16)]
                    msg_v[e, pl.ds(j * 16, 16)] = a * v0 + b * v1
            return carry2

        lax.fori_loop(0, K // 16, group, 0)
        pltpu.sync_copy(msg_v, acc.at[dst_v], add=True)
        return carry

    lax.fori_loop(0, CPT, chunk, 0)
    plsc.subcore_barrier()
    pltpu.sync_copy(acc.at[pl.ds(row0, ROWS_PER_TILE)],
                    part_hbm.at[cid, pl.ds(row0, ROWS_PER_TILE)])

    @pl.when(sid == NS - 1)
    def _():
        pltpu.sync_copy(acc.at[pl.ds(NS * ROWS_PER_TILE, ROWS_TAIL)],
                        part_hbm.at[cid, pl.ds(NS * ROWS_PER_TILE, ROWS_TAIL)])


BLK = 2000


def _tc_prelude_body(x_ref, w_ref, z_ref, curr_ref):
    r = jnp.dot(x_ref[...], w_ref[...], preferred_element_type=jnp.float32)
    z_ref[...] = r[:, :2 * D]
    curr_ref[...] = r[:, 2 * D:]


def _tc_prelude(x, w):
    return pl.pallas_call(
        _tc_prelude_body,
        grid=(N // BLK,),
        in_specs=[pl.BlockSpec((BLK, D), lambda i: (i, 0)),
                  pl.BlockSpec((D, 3 * D), lambda i: (0, 0))],
        out_specs=[pl.BlockSpec((BLK, 2 * D), lambda i: (i, 0)),
                   pl.BlockSpec((BLK, D), lambda i: (i, 0))],
        out_shape=[jax.ShapeDtypeStruct((N, 2 * D), jnp.float32),
                   jax.ShapeDtypeStruct((N, D), jnp.float32)],
    )(x, w)


def _tc_mid_body(curr_ref, part_ref, w_ref, h1_ref, z_ref, curr2_ref):
    h1 = jnp.maximum(curr_ref[...] + part_ref[0] + part_ref[1], 0.0)
    h1_ref[...] = h1
    r = jnp.dot(h1, w_ref[...], preferred_element_type=jnp.float32)
    z_ref[...] = r[:, :2 * D]
    curr2_ref[...] = r[:, 2 * D:]


def _tc_mid(curr, part, w):
    return pl.pallas_call(
        _tc_mid_body,
        grid=(N // BLK,),
        in_specs=[pl.BlockSpec((BLK, D), lambda i: (i, 0)),
                  pl.BlockSpec((NC, BLK, D), lambda i: (0, i, 0)),
                  pl.BlockSpec((D, 3 * D), lambda i: (0, 0))],
        out_specs=[pl.BlockSpec((BLK, D), lambda i: (i, 0)),
                   pl.BlockSpec((BLK, 2 * D), lambda i: (i, 0)),
                   pl.BlockSpec((BLK, D), lambda i: (i, 0))],
        out_shape=[jax.ShapeDtypeStruct((N, D), jnp.float32),
                   jax.ShapeDtypeStruct((N, 2 * D), jnp.float32),
                   jax.ShapeDtypeStruct((N, D), jnp.float32)],
    )(curr, part, w)


def _tc_final_body(h1_ref, curr2_ref, part_ref, srcn_ref, tgtn_ref,
                   g_ref, s_ref, t_ref):
    h2 = jnp.maximum(curr2_ref[...] + part_ref[0] + part_ref[1], 0.0)
    total = jnp.concatenate([h1_ref[...], h2], axis=1)
    col = lax.broadcasted_iota(jnp.int32, (NG, N), 1)
    row = lax.broadcasted_iota(jnp.int32, (NG, N), 0)
    gmask = jnp.where(col // SEG == row, 1.0 / SEG, 0.0)
    g_ref[...] = jnp.dot(gmask, total, preferred_element_type=jnp.float32)
    iota1 = lax.broadcasted_iota(jnp.int32, (1, N), 1)
    ssel = jnp.concatenate(
        [(iota1 == srcn_ref[i]).astype(jnp.float32) for i in range(NG)], axis=0)
    s_ref[...] = jnp.dot(ssel, total, preferred_element_type=jnp.float32)
    tsel = jnp.concatenate(
        [(iota1 == tgtn_ref[i]).astype(jnp.float32) for i in range(NG)], axis=0)
    t_ref[...] = jnp.dot(tsel, total, preferred_element_type=jnp.float32)


def _tc_final(h1, curr2, part, source_node, target_node):
    return pl.pallas_call(
        _tc_final_body,
        in_specs=[pl.BlockSpec(memory_space=pltpu.VMEM),
                  pl.BlockSpec(memory_space=pltpu.VMEM),
                  pl.BlockSpec(memory_space=pltpu.VMEM),
                  pl.BlockSpec(memory_space=pltpu.SMEM),
                  pl.BlockSpec(memory_space=pltpu.SMEM)],
        out_specs=[pl.BlockSpec(memory_space=pltpu.VMEM)] * 3,
        out_shape=[jax.ShapeDtypeStruct((NG, 2 * D), jnp.float32)] * 3,
    )(h1, curr2, part, source_node, target_node)


def kernel(node_feat, graph_sizes, total_target_relation, source_node,
           target_node, total_edge, total_relation_embed, total_relation,
           w0, c0, s0, w1, c1, s1):
    del graph_sizes, total_target_relation, total_relation_embed
    pad = EP - E
    src = jnp.concatenate([total_edge[0], jnp.zeros((pad,), jnp.int32)])
    dst = jnp.concatenate(
        [total_edge[1], N + (jnp.arange(pad, dtype=jnp.int32) % 8)])
    rel = jnp.concatenate([total_relation, jnp.zeros((pad,), jnp.int32)])
    w0p = jnp.concatenate([w0[0], w0[1], s0], axis=1)
    w1p = jnp.concatenate([w1[0], w1[1], s1], axis=1)
    cab0 = c0.T
    cab1 = c1.T
    zero = jnp.zeros((N, D), jnp.float32)
    z1, curr1 = _tc_prelude(node_feat, w0p)
    part1 = _sc_edge_pass(z1, src, dst, rel, cab0, zero)
    h1, z2, curr2 = _tc_mid(curr1, part1, w1p)
    part2 = _sc_edge_pass(z2, src, dst, rel, cab1, zero)
    return _tc_final(h1, curr2, part2, source_node, target_node)


# speedup vs baseline: 1.2795x; 1.2795x over previous
import functools

import jax
import jax.numpy as jnp
from jax import lax
from jax.experimental import pallas as pl
from jax.experimental.pallas import tpu as pltpu
from jax.experimental.pallas import tpu_sc as plsc

N = 10000
E = 160000
D = 128
NG = 8
SEG = N // NG

NC = 2
NS = 16
NW = NC * NS
K = 128
CPT = 40
EP = NW * CPT * K
NCH = EP // K
ACC_ROWS = N + 8
ROWS_PER_TILE = (N // NS) // 8 * 8
ROWS_TAIL = N - NS * ROWS_PER_TILE

_mesh = plsc.VectorSubcoreMesh(
    core_axis_name="c", subcore_axis_name="s", num_cores=NC, num_subcores=NS
)

_dn = lax.GatherDimensionNumbers(
    offset_dims=(), collapsed_slice_dims=(0,), start_index_map=(0,))


def _vgather(vals, idx):
    return lax.gather(vals, idx[:, None], _dn, slice_sizes=(1,),
                      mode=lax.GatherScatterMode.PROMISE_IN_BOUNDS)


@functools.partial(
    pl.kernel,
    out_type=jax.ShapeDtypeStruct((NC, N, D), jnp.float32),
    mesh=_mesh,
    scratch_types=[
        pltpu.VMEM((K,), jnp.int32),
        pltpu.VMEM((K,), jnp.int32),
        pltpu.VMEM((K,), jnp.int32),
        pltpu.VMEM((K, 2 * D), jnp.float32),
        pltpu.VMEM((K, D), jnp.float32),
        pltpu.VMEM((16,), jnp.float32),
        pltpu.VMEM((16,), jnp.float32),
        pltpu.VMEM_SHARED((ACC_ROWS, D), jnp.float32),
        pltpu.SemaphoreType.DMA,
    ],
)
def _sc_edge_pass(z_hbm, src_hbm, dst_hbm, rel_hbm, cab_hbm, zero_hbm,
                  part_hbm, src_v, dst_v, rel_v, rows_v, msg_v, ca_v, cb_v,
                  acc, sem):
    cid = lax.axis_index("c")
    sid = lax.axis_index("s")
    wid = sid * NC + cid
    pltpu.sync_copy(cab_hbm.at[0], ca_v)
    pltpu.sync_copy(cab_hbm.at[1], cb_v)
    row0 = sid * ROWS_PER_TILE
    pltpu.sync_copy(zero_hbm.at[pl.ds(row0, ROWS_PER_TILE)],
                    acc.at[pl.ds(row0, ROWS_PER_TILE)])

    @pl.when(sid == NS - 1)
    def _():
        pltpu.sync_copy(zero_hbm.at[pl.ds(NS * ROWS_PER_TILE, ROWS_TAIL)],
                        acc.at[pl.ds(NS * ROWS_PER_TILE, ROWS_TAIL)])

    plsc.subcore_barrier()

    def chunk(i, carry):
        c = i * NW + wid

        @pl.when(c < E // K)
        def _():
            eb = c * K
            pltpu.sync_copy(src_hbm.at[pl.ds(eb, K)], src_v)
            pltpu.sync_copy(dst_hbm.at[pl.ds(eb, K)], dst_v)
            pltpu.sync_copy(rel_hbm.at[pl.ds(eb, K)], rel_v)
            pltpu.async_copy(z_hbm.at[src_v], rows_v, sem).wait()

            def group(g, carry2):
                rel_vec = rel_v[pl.ds(g * 16, 16)]
                a_vec = _vgather(ca_v[...], rel_vec)
                b_vec = _vgather(cb_v[...], rel_vec)
                for k in range(16):
                    a = a_vec[k]
                    b = b_vec[k]
                    e = g * 16 + k
                    for j in range(D // 16):
                        v0 = rows_v[e, pl.ds(j * 16, 16)]
                        v1 = rows_v[e, pl.ds(D + j * 16, 16)]
                        msg_v[e, pl.ds(j * 16, 16)] = a * v0 + b * v1
                return carry2

            lax.fori_loop(0, K // 16, group, 0)
            pltpu.sync_copy(msg_v, acc.at[dst_v], add=True)

        return carry

    lax.fori_loop(0, CPT, chunk, 0)
    plsc.subcore_barrier()
    pltpu.sync_copy(acc.at[pl.ds(row0, ROWS_PER_TILE)],
                    part_hbm.at[cid, pl.ds(row0, ROWS_PER_TILE)])

    @pl.when(sid == NS - 1)
    def _():
        pltpu.sync_copy(acc.at[pl.ds(NS * ROWS_PER_TILE, ROWS_TAIL)],
                        part_hbm.at[cid, pl.ds(NS * ROWS_PER_TILE, ROWS_TAIL)])


BLK = 2000


def _tc_prelude_body(x_ref, w_ref, z_ref, curr_ref):
    r = jnp.dot(x_ref[...], w_ref[...], preferred_element_type=jnp.float32)
    z_ref[...] = r[:, :2 * D]
    curr_ref[...] = r[:, 2 * D:]


def _tc_prelude(x, w):
    return pl.pallas_call(
        _tc_prelude_body,
        grid=(N // BLK,),
        in_specs=[pl.BlockSpec((BLK, D), lambda i: (i, 0)),
                  pl.BlockSpec((D, 3 * D), lambda i: (0, 0))],
        out_specs=[pl.BlockSpec((BLK, 2 * D), lambda i: (i, 0)),
                   pl.BlockSpec((BLK, D), lambda i: (i, 0))],
        out_shape=[jax.ShapeDtypeStruct((N, 2 * D), jnp.float32),
                   jax.ShapeDtypeStruct((N, D), jnp.float32)],
    )(x, w)


def _tc_mid_body(curr_ref, part_ref, w_ref, h1_ref, z_ref, curr2_ref):
    h1 = jnp.maximum(curr_ref[...] + part_ref[0] + part_ref[1], 0.0)
    h1_ref[...] = h1
    r = jnp.dot(h1, w_ref[...], preferred_element_type=jnp.float32)
    z_ref[...] = r[:, :2 * D]
    curr2_ref[...] = r[:, 2 * D:]


def _tc_mid(curr, part, w):
    return pl.pallas_call(
        _tc_mid_body,
        grid=(N // BLK,),
        in_specs=[pl.BlockSpec((BLK, D), lambda i: (i, 0)),
                  pl.BlockSpec((NC, BLK, D), lambda i: (0, i, 0)),
                  pl.BlockSpec((D, 3 * D), lambda i: (0, 0))],
        out_specs=[pl.BlockSpec((BLK, D), lambda i: (i, 0)),
                   pl.BlockSpec((BLK, 2 * D), lambda i: (i, 0)),
                   pl.BlockSpec((BLK, D), lambda i: (i, 0))],
        out_shape=[jax.ShapeDtypeStruct((N, D), jnp.float32),
                   jax.ShapeDtypeStruct((N, 2 * D), jnp.float32),
                   jax.ShapeDtypeStruct((N, D), jnp.float32)],
    )(curr, part, w)


def _tc_final_body(h1_ref, curr2_ref, part_ref, srcn_ref, tgtn_ref,
                   g_ref, s_ref, t_ref):
    h2 = jnp.maximum(curr2_ref[...] + part_ref[0] + part_ref[1], 0.0)
    total = jnp.concatenate([h1_ref[...], h2], axis=1)
    col = lax.broadcasted_iota(jnp.int32, (NG, N), 1)
    row = lax.broadcasted_iota(jnp.int32, (NG, N), 0)
    gmask = jnp.where(col // SEG == row, 1.0 / SEG, 0.0)
    g_ref[...] = jnp.dot(gmask, total, preferred_element_type=jnp.float32)
    iota1 = lax.broadcasted_iota(jnp.int32, (1, N), 1)
    ssel = jnp.concatenate(
        [(iota1 == srcn_ref[i]).astype(jnp.float32) for i in range(NG)], axis=0)
    s_ref[...] = jnp.dot(ssel, total, preferred_element_type=jnp.float32)
    tsel = jnp.concatenate(
        [(iota1 == tgtn_ref[i]).astype(jnp.float32) for i in range(NG)], axis=0)
    t_ref[...] = jnp.dot(tsel, total, preferred_element_type=jnp.float32)


def _tc_final(h1, curr2, part, source_node, target_node):
    return pl.pallas_call(
        _tc_final_body,
        in_specs=[pl.BlockSpec(memory_space=pltpu.VMEM),
                  pl.BlockSpec(memory_space=pltpu.VMEM),
                  pl.BlockSpec(memory_space=pltpu.VMEM),
                  pl.BlockSpec(memory_space=pltpu.SMEM),
                  pl.BlockSpec(memory_space=pltpu.SMEM)],
        out_specs=[pl.BlockSpec(memory_space=pltpu.VMEM)] * 3,
        out_shape=[jax.ShapeDtypeStruct((NG, 2 * D), jnp.float32)] * 3,
    )(h1, curr2, part, source_node, target_node)


def kernel(node_feat, graph_sizes, total_target_relation, source_node,
           target_node, total_edge, total_relation_embed, total_relation,
           w0, c0, s0, w1, c1, s1):
    del graph_sizes, total_target_relation, total_relation_embed
    pad = EP - E
    src = jnp.concatenate([total_edge[0], jnp.zeros((pad,), jnp.int32)])
    dst = jnp.concatenate(
        [total_edge[1], N + (jnp.arange(pad, dtype=jnp.int32) % 8)])
    rel = jnp.concatenate([total_relation, jnp.zeros((pad,), jnp.int32)])
    w0p = jnp.concatenate([w0[0], w0[1], s0], axis=1)
    w1p = jnp.concatenate([w1[0], w1[1], s1], axis=1)
    cab0 = c0.T
    cab1 = c1.T
    zero = jnp.zeros((N, D), jnp.float32)
    z1, curr1 = _tc_prelude(node_feat, w0p)
    part1 = _sc_edge_pass(z1, src, dst, rel, cab0, zero)
    h1, z2, curr2 = _tc_mid(curr1, part1, w1p)
    part2 = _sc_edge_pass(z2, src, dst, rel, cab1, zero)
    return _tc_final(h1, curr2, part2, source_node, target_node)


# speedup vs baseline: 1.3507x; 1.0556x over previous
import functools

import jax
import jax.numpy as jnp
from jax import lax
from jax.experimental import pallas as pl
from jax.experimental.pallas import tpu as pltpu
from jax.experimental.pallas import tpu_sc as plsc

N = 10000
E = 160000
D = 128
NG = 8
SEG = N // NG

NC = 2
NS = 16
NW = NC * NS
K = 64
NCH = E // K
PAIRS = (NCH + 2 * NW - 1) // (2 * NW)
ACC_ROWS = N
ROWS_PER_TILE = (N // NS) // 8 * 8
ROWS_TAIL = N - NS * ROWS_PER_TILE

_mesh = plsc.VectorSubcoreMesh(
    core_axis_name="c", subcore_axis_name="s", num_cores=NC, num_subcores=NS
)

_dn = lax.GatherDimensionNumbers(
    offset_dims=(), collapsed_slice_dims=(0,), start_index_map=(0,))


def _vgather(vals, idx):
    return lax.gather(vals, idx[:, None], _dn, slice_sizes=(1,),
                      mode=lax.GatherScatterMode.PROMISE_IN_BOUNDS)


@functools.partial(
    pl.kernel,
    out_type=jax.ShapeDtypeStruct((NC, N, D), jnp.float32),
    mesh=_mesh,
    scratch_types=[
        pltpu.VMEM((K,), jnp.int32),
        pltpu.VMEM((K,), jnp.int32),
        pltpu.VMEM((K,), jnp.int32),
        pltpu.VMEM((K, 2 * D), jnp.float32),
        pltpu.VMEM((K, D), jnp.float32),
        pltpu.VMEM((K,), jnp.int32),
        pltpu.VMEM((K,), jnp.int32),
        pltpu.VMEM((K,), jnp.int32),
        pltpu.VMEM((K, 2 * D), jnp.float32),
        pltpu.VMEM((K, D), jnp.float32),
        pltpu.VMEM((16,), jnp.float32),
        pltpu.VMEM((16,), jnp.float32),
        pltpu.VMEM_SHARED((ACC_ROWS, D), jnp.float32),
        pltpu.SemaphoreType.DMA,
        pltpu.SemaphoreType.DMA,
    ],
)
def _sc_edge_pass(z_hbm, src_hbm, dst_hbm, rel_hbm, cab_hbm, zero_hbm,
                  part_hbm, src0_v, dst0_v, rel0_v, rows0_v, msg0_v,
                  src1_v, dst1_v, rel1_v, rows1_v, msg1_v, ca_v, cb_v,
                  acc, sem0, sem1):
    cid = lax.axis_index("c")
    sid = lax.axis_index("s")
    wid = sid * NC + cid
    pltpu.sync_copy(cab_hbm.at[0], ca_v)
    pltpu.sync_copy(cab_hbm.at[1], cb_v)
    row0 = sid * ROWS_PER_TILE
    pltpu.sync_copy(zero_hbm.at[pl.ds(row0, ROWS_PER_TILE)],
                    acc.at[pl.ds(row0, ROWS_PER_TILE)])

    @pl.when(sid == NS - 1)
    def _():
        pltpu.sync_copy(zero_hbm.at[pl.ds(NS * ROWS_PER_TILE, ROWS_TAIL)],
                        acc.at[pl.ds(NS * ROWS_PER_TILE, ROWS_TAIL)])

    plsc.subcore_barrier()

    def load_idx(c, src_v, dst_v, rel_v):
        eb = c * K
        pltpu.sync_copy(src_hbm.at[pl.ds(eb, K)], src_v)
        pltpu.sync_copy(dst_hbm.at[pl.ds(eb, K)], dst_v)
        pltpu.sync_copy(rel_hbm.at[pl.ds(eb, K)], rel_v)

    def compute_scatter(rel_v, dst_v, rows_v, msg_v):
        def group(g, carry2):
            rel_vec = rel_v[pl.ds(g * 16, 16)]
            a_vec = _vgather(ca_v[...], rel_vec)
            b_vec = _vgather(cb_v[...], rel_vec)
            for k in range(16):
                a = a_vec[k]
                b = b_vec[k]
                e = g * 16 + k
                for j in range(D // 16):
                    v0 = rows_v[e, pl.ds(j * 16, 16)]
                    v1 = rows_v[e, pl.ds(D + j * 16, 16)]
                    msg_v[e, pl.ds(j * 16, 16)] = a * v0 + b * v1
            return carry2

        lax.fori_loop(0, K // 16, group, 0)
        pltpu.sync_copy(msg_v, acc.at[dst_v], add=True)

    load_idx(wid, src0_v, dst0_v, rel0_v)
    pltpu.async_copy(z_hbm.at[src0_v], rows0_v, sem0)

    def pair(p, carry):
        c0 = (2 * p) * NW + wid
        c1 = c0 + NW

        @pl.when(c1 < NCH)
        def _():
            load_idx(c1, src1_v, dst1_v, rel1_v)
            pltpu.async_copy(z_hbm.at[src1_v], rows1_v, sem1)

        @pl.when(c0 < NCH)
        def _():
            pltpu.make_async_copy(z_hbm.at[src0_v], rows0_v, sem0).wait()
            compute_scatter(rel0_v, dst0_v, rows0_v, msg0_v)

        @pl.when(c0 + 2 * NW < NCH)
        def _():
            load_idx(c0 + 2 * NW, src0_v, dst0_v, rel0_v)
            pltpu.async_copy(z_hbm.at[src0_v], rows0_v, sem0)

        @pl.when(c1 < NCH)
        def _():
            pltpu.make_async_copy(z_hbm.at[src1_v], rows1_v, sem1).wait()
            compute_scatter(rel1_v, dst1_v, rows1_v, msg1_v)

        return carry

    lax.fori_loop(0, PAIRS, pair, 0)
    plsc.subcore_barrier()
    pltpu.sync_copy(acc.at[pl.ds(row0, ROWS_PER_TILE)],
                    part_hbm.at[cid, pl.ds(row0, ROWS_PER_TILE)])

    @pl.when(sid == NS - 1)
    def _():
        pltpu.sync_copy(acc.at[pl.ds(NS * ROWS_PER_TILE, ROWS_TAIL)],
                        part_hbm.at[cid, pl.ds(NS * ROWS_PER_TILE, ROWS_TAIL)])


BLK = 2000


def _tc_prelude_body(x_ref, w_ref, z_ref, curr_ref):
    r = jnp.dot(x_ref[...], w_ref[...], preferred_element_type=jnp.float32)
    z_ref[...] = r[:, :2 * D]
    curr_ref[...] = r[:, 2 * D:]


def _tc_prelude(x, w):
    return pl.pallas_call(
        _tc_prelude_body,
        grid=(N // BLK,),
        in_specs=[pl.BlockSpec((BLK, D), lambda i: (i, 0)),
                  pl.BlockSpec((D, 3 * D), lambda i: (0, 0))],
        out_specs=[pl.BlockSpec((BLK, 2 * D), lambda i: (i, 0)),
                   pl.BlockSpec((BLK, D), lambda i: (i, 0))],
        out_shape=[jax.ShapeDtypeStruct((N, 2 * D), jnp.float32),
                   jax.ShapeDtypeStruct((N, D), jnp.float32)],
    )(x, w)


def _tc_mid_body(curr_ref, part_ref, w_ref, h1_ref, z_ref, curr2_ref):
    h1 = jnp.maximum(curr_ref[...] + part_ref[0] + part_ref[1], 0.0)
    h1_ref[...] = h1
    r = jnp.dot(h1, w_ref[...], preferred_element_type=jnp.float32)
    z_ref[...] = r[:, :2 * D]
    curr2_ref[...] = r[:, 2 * D:]


def _tc_mid(curr, part, w):
    return pl.pallas_call(
        _tc_mid_body,
        grid=(N // BLK,),
        in_specs=[pl.BlockSpec((BLK, D), lambda i: (i, 0)),
                  pl.BlockSpec((NC, BLK, D), lambda i: (0, i, 0)),
                  pl.BlockSpec((D, 3 * D), lambda i: (0, 0))],
        out_specs=[pl.BlockSpec((BLK, D), lambda i: (i, 0)),
                   pl.BlockSpec((BLK, 2 * D), lambda i: (i, 0)),
                   pl.BlockSpec((BLK, D), lambda i: (i, 0))],
        out_shape=[jax.ShapeDtypeStruct((N, D), jnp.float32),
                   jax.ShapeDtypeStruct((N, 2 * D), jnp.float32),
                   jax.ShapeDtypeStruct((N, D), jnp.float32)],
    )(curr, part, w)


def _tc_final_body(h1_ref, curr2_ref, part_ref, srcn_ref, tgtn_ref,
                   g_ref, s_ref, t_ref):
    h2 = jnp.maximum(curr2_ref[...] + part_ref[0] + part_ref[1], 0.0)
    total = jnp.concatenate([h1_ref[...], h2], axis=1)
    col = lax.broadcasted_iota(jnp.int32, (NG, N), 1)
    row = lax.broadcasted_iota(jnp.int32, (NG, N), 0)
    gmask = jnp.where(col // SEG == row, 1.0 / SEG, 0.0)
    g_ref[...] = jnp.dot(gmask, total, preferred_element_type=jnp.float32)
    iota1 = lax.broadcasted_iota(jnp.int32, (1, N), 1)
    ssel = jnp.concatenate(
        [(iota1 == srcn_ref[i]).astype(jnp.float32) for i in range(NG)], axis=0)
    s_ref[...] = jnp.dot(ssel, total, preferred_element_type=jnp.float32)
    tsel = jnp.concatenate(
        [(iota1 == tgtn_ref[i]).astype(jnp.float32) for i in range(NG)], axis=0)
    t_ref[...] = jnp.dot(tsel, total, preferred_element_type=jnp.float32)


def _tc_final(h1, curr2, part, source_node, target_node):
    return pl.pallas_call(
        _tc_final_body,
        in_specs=[pl.BlockSpec(memory_space=pltpu.VMEM),
                  pl.BlockSpec(memory_space=pltpu.VMEM),
                  pl.BlockSpec(memory_space=pltpu.VMEM),
                  pl.BlockSpec(memory_space=pltpu.SMEM),
                  pl.BlockSpec(memory_space=pltpu.SMEM)],
        out_specs=[pl.BlockSpec(memory_space=pltpu.VMEM)] * 3,
        out_shape=[jax.ShapeDtypeStruct((NG, 2 * D), jnp.float32)] * 3,
    )(h1, curr2, part, source_node, target_node)


def kernel(node_feat, graph_sizes, total_target_relation, source_node,
           target_node, total_edge, total_relation_embed, total_relation,
           w0, c0, s0, w1, c1, s1):
    del graph_sizes, total_target_relation, total_relation_embed
    src = total_edge[0]
    dst = total_edge[1]
    rel = total_relation
    w0p = jnp.concatenate([w0[0], w0[1], s0], axis=1)
    w1p = jnp.concatenate([w1[0], w1[1], s1], axis=1)
    cab0 = c0.T
    cab1 = c1.T
    zero = jnp.zeros((N, D), jnp.float32)
    z1, curr1 = _tc_prelude(node_feat, w0p)
    part1 = _sc_edge_pass(z1, src, dst, rel, cab0, zero)
    h1, z2, curr2 = _tc_mid(curr1, part1, w1p)
    part2 = _sc_edge_pass(z2, src, dst, rel, cab1, zero)
    return _tc_final(h1, curr2, part2, source_node, target_node)


# speedup vs baseline: 3.3641x; 2.4907x over previous
import functools

import jax
import jax.numpy as jnp
from jax import lax
from jax.experimental import pallas as pl
from jax.experimental.pallas import tpu as pltpu
from jax.experimental.pallas import tpu_sc as plsc

N = 10000
E = 160000
D = 128
NG = 8
SEG = N // NG

NC = 2
NS = 16
NW = NC * NS
K = 128
NCH = E // K
PAIRS = (NCH + 2 * NW - 1) // (2 * NW)
ACC_ROWS = N
ROWS_PER_TILE = (N // NS) // 8 * 8
ROWS_TAIL = N - NS * ROWS_PER_TILE

_mesh = plsc.VectorSubcoreMesh(
    core_axis_name="c", subcore_axis_name="s", num_cores=NC, num_subcores=NS
)

_dn = lax.GatherDimensionNumbers(
    offset_dims=(), collapsed_slice_dims=(0,), start_index_map=(0,))


def _vgather(vals, idx):
    return lax.gather(vals, idx[:, None], _dn, slice_sizes=(1,),
                      mode=lax.GatherScatterMode.PROMISE_IN_BOUNDS)


@functools.partial(
    pl.kernel,
    out_type=jax.ShapeDtypeStruct((NC, N, D), jnp.float32),
    mesh=_mesh,
    scratch_types=[
        pltpu.VMEM((K,), jnp.int32),
        pltpu.VMEM((K,), jnp.int32),
        pltpu.VMEM((K,), jnp.int32),
        pltpu.VMEM((K, D), jnp.int32),
        pltpu.VMEM((K,), jnp.int32),
        pltpu.VMEM((K,), jnp.int32),
        pltpu.VMEM((K,), jnp.int32),
        pltpu.VMEM((K, D), jnp.int32),
        pltpu.VMEM((K, D), jnp.float32),
        pltpu.VMEM((16,), jnp.float32),
        pltpu.VMEM((16,), jnp.float32),
        pltpu.VMEM_SHARED((ACC_ROWS, D), jnp.float32),
        pltpu.SemaphoreType.DMA,
        pltpu.SemaphoreType.DMA,
    ],
)
def _sc_edge_pass(z_hbm, src_hbm, dst_hbm, rel_hbm, cab_hbm, zero_hbm,
                  part_hbm, src0_v, dst0_v, rel0_v, rows0_v,
                  src1_v, dst1_v, rel1_v, rows1_v, msg_v, ca_v, cb_v,
                  acc, sem0, sem1):
    cid = lax.axis_index("c")
    sid = lax.axis_index("s")
    wid = sid * NC + cid
    pltpu.sync_copy(cab_hbm.at[0], ca_v)
    pltpu.sync_copy(cab_hbm.at[1], cb_v)
    row0 = sid * ROWS_PER_TILE
    pltpu.sync_copy(zero_hbm.at[pl.ds(row0, ROWS_PER_TILE)],
                    acc.at[pl.ds(row0, ROWS_PER_TILE)])

    @pl.when(sid == NS - 1)
    def _():
        pltpu.sync_copy(zero_hbm.at[pl.ds(NS * ROWS_PER_TILE, ROWS_TAIL)],
                        acc.at[pl.ds(NS * ROWS_PER_TILE, ROWS_TAIL)])

    plsc.subcore_barrier()

    def load_idx(c, src_v, dst_v, rel_v):
        eb = c * K
        pltpu.sync_copy(src_hbm.at[pl.ds(eb, K)], src_v)
        pltpu.sync_copy(dst_hbm.at[pl.ds(eb, K)], dst_v)
        pltpu.sync_copy(rel_hbm.at[pl.ds(eb, K)], rel_v)

    def compute_scatter(rel_v, dst_v, rows_v):
        def group(g, carry2):
            rel_vec = rel_v[pl.ds(g * 16, 16)]
            a_vec = _vgather(ca_v[...], rel_vec)
            b_vec = _vgather(cb_v[...], rel_vec)
            for k in range(16):
                a = a_vec[k]
                b = b_vec[k]
                e = g * 16 + k
                for j in range(D // 16):
                    xi = rows_v[e, pl.ds(j * 16, 16)]
                    v0 = lax.bitcast_convert_type(xi << 16, jnp.float32)
                    v1 = lax.bitcast_convert_type(
                        xi & jnp.int32(-65536), jnp.float32)
                    msg_v[e, pl.ds(j * 16, 16)] = a * v0 + b * v1
            return carry2

        lax.fori_loop(0, K // 16, group, 0)
        pltpu.sync_copy(msg_v, acc.at[dst_v], add=True)

    load_idx(wid, src0_v, dst0_v, rel0_v)
    pltpu.async_copy(z_hbm.at[src0_v], rows0_v, sem0)

    def pair(p, carry):
        c0 = (2 * p) * NW + wid
        c1 = c0 + NW

        @pl.when(c1 < NCH)
        def _():
            load_idx(c1, src1_v, dst1_v, rel1_v)
            pltpu.async_copy(z_hbm.at[src1_v], rows1_v, sem1)

        @pl.when(c0 < NCH)
        def _():
            pltpu.make_async_copy(z_hbm.at[src0_v], rows0_v, sem0).wait()
            compute_scatter(rel0_v, dst0_v, rows0_v)

        @pl.when(c0 + 2 * NW < NCH)
        def _():
            load_idx(c0 + 2 * NW, src0_v, dst0_v, rel0_v)
            pltpu.async_copy(z_hbm.at[src0_v], rows0_v, sem0)

        @pl.when(c1 < NCH)
        def _():
            pltpu.make_async_copy(z_hbm.at[src1_v], rows1_v, sem1).wait()
            compute_scatter(rel1_v, dst1_v, rows1_v)

        return carry

    lax.fori_loop(0, PAIRS, pair, 0)
    plsc.subcore_barrier()
    pltpu.sync_copy(acc.at[pl.ds(row0, ROWS_PER_TILE)],
                    part_hbm.at[cid, pl.ds(row0, ROWS_PER_TILE)])

    @pl.when(sid == NS - 1)
    def _():
        pltpu.sync_copy(acc.at[pl.ds(NS * ROWS_PER_TILE, ROWS_TAIL)],
                        part_hbm.at[cid, pl.ds(NS * ROWS_PER_TILE, ROWS_TAIL)])


BLK = 2000


def _pack_bf16_pair(y0, y1):
    b0 = lax.bitcast_convert_type(y0, jnp.int32) + jnp.int32(0x8000)
    b1 = lax.bitcast_convert_type(y1, jnp.int32) + jnp.int32(0x8000)
    return (b1 & jnp.int32(-65536)) | ((b0 >> 16) & jnp.int32(0xFFFF))


def _tc_prelude_body(x_ref, w_ref, z_ref, curr_ref):
    r = jnp.dot(x_ref[...], w_ref[...], preferred_element_type=jnp.float32)
    z_ref[...] = _pack_bf16_pair(r[:, :D], r[:, D:2 * D])
    curr_ref[...] = r[:, 2 * D:]


def _tc_prelude(x, w):
    return pl.pallas_call(
        _tc_prelude_body,
        grid=(N // BLK,),
        in_specs=[pl.BlockSpec((BLK, D), lambda i: (i, 0)),
                  pl.BlockSpec((D, 3 * D), lambda i: (0, 0))],
        out_specs=[pl.BlockSpec((BLK, D), lambda i: (i, 0)),
                   pl.BlockSpec((BLK, D), lambda i: (i, 0))],
        out_shape=[jax.ShapeDtypeStruct((N, D), jnp.int32),
                   jax.ShapeDtypeStruct((N, D), jnp.float32)],
    )(x, w)


def _tc_mid_body(curr_ref, part_ref, w_ref, h1_ref, z_ref, curr2_ref):
    h1 = jnp.maximum(curr_ref[...] + part_ref[0] + part_ref[1], 0.0)
    h1_ref[...] = h1
    r = jnp.dot(h1, w_ref[...], preferred_element_type=jnp.float32)
    z_ref[...] = _pack_bf16_pair(r[:, :D], r[:, D:2 * D])
    curr2_ref[...] = r[:, 2 * D:]


def _tc_mid(curr, part, w):
    return pl.pallas_call(
        _tc_mid_body,
        grid=(N // BLK,),
        in_specs=[pl.BlockSpec((BLK, D), lambda i: (i, 0)),
                  pl.BlockSpec((NC, BLK, D), lambda i: (0, i, 0)),
                  pl.BlockSpec((D, 3 * D), lambda i: (0, 0))],
        out_specs=[pl.BlockSpec((BLK, D), lambda i: (i, 0)),
                   pl.BlockSpec((BLK, D), lambda i: (i, 0)),
                   pl.BlockSpec((BLK, D), lambda i: (i, 0))],
        out_shape=[jax.ShapeDtypeStruct((N, D), jnp.float32),
                   jax.ShapeDtypeStruct((N, D), jnp.int32),
                   jax.ShapeDtypeStruct((N, D), jnp.float32)],
    )(curr, part, w)


def _tc_final_body(h1_ref, curr2_ref, part_ref, srcn_ref, tgtn_ref,
                   g_ref, s_ref, t_ref):
    h2 = jnp.maximum(curr2_ref[...] + part_ref[0] + part_ref[1], 0.0)
    total = jnp.concatenate([h1_ref[...], h2], axis=1)
    col = lax.broadcasted_iota(jnp.int32, (NG, N), 1)
    row = lax.broadcasted_iota(jnp.int32, (NG, N), 0)
    gmask = jnp.where(col // SEG == row, 1.0 / SEG, 0.0)
    g_ref[...] = jnp.dot(gmask, total, preferred_element_type=jnp.float32)
    iota1 = lax.broadcasted_iota(jnp.int32, (1, N), 1)
    ssel = jnp.concatenate(
        [(iota1 == srcn_ref[i]).astype(jnp.float32) for i in range(NG)], axis=0)
    s_ref[...] = jnp.dot(ssel, total, preferred_element_type=jnp.float32)
    tsel = jnp.concatenate(
        [(iota1 == tgtn_ref[i]).astype(jnp.float32) for i in range(NG)], axis=0)
    t_ref[...] = jnp.dot(tsel, total, preferred_element_type=jnp.float32)


def _tc_final(h1, curr2, part, source_node, target_node):
    return pl.pallas_call(
        _tc_final_body,
        in_specs=[pl.BlockSpec(memory_space=pltpu.VMEM),
                  pl.BlockSpec(memory_space=pltpu.VMEM),
                  pl.BlockSpec(memory_space=pltpu.VMEM),
                  pl.BlockSpec(memory_space=pltpu.SMEM),
                  pl.BlockSpec(memory_space=pltpu.SMEM)],
        out_specs=[pl.BlockSpec(memory_space=pltpu.VMEM)] * 3,
        out_shape=[jax.ShapeDtypeStruct((NG, 2 * D), jnp.float32)] * 3,
    )(h1, curr2, part, source_node, target_node)


def kernel(node_feat, graph_sizes, total_target_relation, source_node,
           target_node, total_edge, total_relation_embed, total_relation,
           w0, c0, s0, w1, c1, s1):
    del graph_sizes, total_target_relation, total_relation_embed
    src = total_edge[0]
    dst = total_edge[1]
    rel = total_relation
    w0p = jnp.concatenate([w0[0], w0[1], s0], axis=1)
    w1p = jnp.concatenate([w1[0], w1[1], s1], axis=1)
    cab0 = c0.T
    cab1 = c1.T
    zero = jnp.zeros((N, D), jnp.float32)
    z1, curr1 = _tc_prelude(node_feat, w0p)
    part1 = _sc_edge_pass(z1, src, dst, rel, cab0, zero)
    h1, z2, curr2 = _tc_mid(curr1, part1, w1p)
    part2 = _sc_edge_pass(z2, src, dst, rel, cab1, zero)
    return _tc_final(h1, curr2, part2, source_node, target_node)
